# Initial kernel scaffold; baseline (speedup 1.0000x reference)
import jax
import jax.numpy as jnp
from jax.experimental import pallas as pl


def kernel(x, edge_index, W1, a_src1, a_dst1, b1, W2, a_src2, a_dst2, b2, Wr, br, Wc, bc):
    raise NotImplementedError("write your pallas kernel here")


# speedup vs baseline: 29.1517x; 29.1517x over previous
import functools

import jax
import jax.numpy as jnp
from jax import lax
from jax.experimental import pallas as pl
from jax.experimental.pallas import tpu as pltpu
from jax.experimental.pallas import tpu_sc as plsc

N = 10000
E = 320000
D = 128
H = 64
C = 2

NC = 2
NS = 16
NW = NC * NS
NP = 10240
EPT = 10240
EP = EPT * NW
K = 128
NCH = EPT // K
NSL = NP // NS


def _sc_gat_aggregate(h, asv, adv, srcr, dstr):
    mesh = plsc.VectorSubcoreMesh(core_axis_name="c", subcore_axis_name="s")

    @functools.partial(
        pl.kernel,
        out_type=(
            jax.ShapeDtypeStruct((NC, NP, H), jnp.float32),
            jax.ShapeDtypeStruct((NC, NP), jnp.float32),
        ),
        mesh=mesh,
        compiler_params=pltpu.CompilerParams(
            needs_layout_passes=False, use_tc_tiling_on_sc=False),
        scratch_types=[
            pltpu.VMEM((NCH, K), jnp.int32),
            pltpu.VMEM((NCH, K), jnp.int32),
            pltpu.VMEM((N,), jnp.float32),
            pltpu.VMEM((N,), jnp.float32),
            pltpu.VMEM((NCH, K), jnp.float32),
            pltpu.VMEM((2, K, H), jnp.float32),
            pltpu.VMEM((NSL,), jnp.float32),
            pltpu.VMEM_SHARED((NP, H), jnp.float32),
            pltpu.VMEM_SHARED((NP,), jnp.float32),
            pltpu.SemaphoreType.DMA,
        ],
    )
    def k(h_hbm, as_hbm, ad_hbm, src_hbm, dst_hbm,
          out_hbm, den_hbm,
          src_v, dst_v, as_v, ad_v, ex_v, rows_v, zden_v, acc_s, den_s, gsem):
        core = lax.axis_index("c")
        sid = lax.axis_index("s")
        wid = core * NS + sid

        pltpu.sync_copy(src_hbm.at[wid], src_v)
        pltpu.sync_copy(dst_hbm.at[wid], dst_v)
        pltpu.sync_copy(as_hbm, as_v)
        pltpu.sync_copy(ad_hbm, ad_v)

        z16 = jnp.zeros((16,), jnp.float32)

        def zrow(kk, carry):
            for j in range(H // 16):
                rows_v[0, kk, pl.ds(j * 16, 16)] = z16
            return carry
        lax.fori_loop(0, K, zrow, 0)

        def zden(i, carry):
            zden_v[pl.ds(i * 16, 16)] = z16
            return carry
        lax.fori_loop(0, NSL // 16, zden, 0)

        for q in range(NSL // K):
            pltpu.sync_copy(rows_v.at[0], acc_s.at[pl.ds(sid * NSL + q * K, K)])
        pltpu.sync_copy(zden_v, den_s.at[pl.ds(sid * NSL, NSL)])

        base = wid * EPT
        iota = lax.iota(jnp.int32, 16)

        def exbody(c, carry):
            for j in range(K // 16):
                s16 = src_v[c, pl.ds(j * 16, 16)]
                d16 = dst_v[c, pl.ds(j * 16, 16)]
                av = plsc.load_gather(as_v, [s16])
                bv = plsc.load_gather(ad_v, [d16])
                e = av + bv
                e = jnp.maximum(e, e * 0.2)
                ex = jnp.exp(e)
                gid = base + c * K + j * 16 + iota
                ex = jnp.where(gid < E, ex, 0.0)
                ex_v[c, pl.ds(j * 16, 16)] = ex
            return carry
        lax.fori_loop(0, NCH, exbody, 0)

        plsc.subcore_barrier()

        pltpu.async_copy(h_hbm.at[src_v.at[0]], rows_v.at[0], gsem)

        def chunk(c, carry):
            p = lax.rem(c, 2)
            pltpu.make_async_copy(h_hbm.at[src_v.at[c]], rows_v.at[p], gsem).wait()

            @pl.when(c < NCH - 1)
            def _():
                pltpu.async_copy(h_hbm.at[src_v.at[c + 1]], rows_v.at[1 - p], gsem)

            def scale(kk, carry2):
                exb = plsc.load_gather(
                    ex_v, [jnp.full((16,), c, jnp.int32),
                           jnp.full((16,), kk, jnp.int32)])
                for j in range(H // 16):
                    rows_v[p, kk, pl.ds(j * 16, 16)] = (
                        rows_v[p, kk, pl.ds(j * 16, 16)] * exb)
                return carry2
            lax.fori_loop(0, K, scale, 0)

            pltpu.sync_copy(rows_v.at[p], acc_s.at[dst_v.at[c]], add=True)
            pltpu.sync_copy(ex_v.at[c], den_s.at[dst_v.at[c]], add=True)
            return carry
        lax.fori_loop(0, NCH, chunk, 0)

        plsc.subcore_barrier()

        pltpu.sync_copy(acc_s.at[pl.ds(sid * NSL, NSL)],
                        out_hbm.at[core, pl.ds(sid * NSL, NSL)])
        pltpu.sync_copy(den_s.at[pl.ds(sid * NSL, NSL)],
                        den_hbm.at[core, pl.ds(sid * NSL, NSL)])

    return k(h, asv, adv, srcr, dstr)


def _tc_pre(x, W1, a_s, a_d):
    def body(x_ref, w_ref, as_ref, ad_ref, h_ref, s_ref, d_ref):
        h = jnp.dot(x_ref[...], w_ref[...], preferred_element_type=jnp.float32)
        h_ref[...] = h
        s_ref[...] = jnp.dot(h, as_ref[...], preferred_element_type=jnp.float32)
        d_ref[...] = jnp.dot(h, ad_ref[...], preferred_element_type=jnp.float32)

    return pl.pallas_call(
        body,
        out_shape=(
            jax.ShapeDtypeStruct((N, H), jnp.float32),
            jax.ShapeDtypeStruct((N, 1), jnp.float32),
            jax.ShapeDtypeStruct((N, 1), jnp.float32),
        ),
    )(x, W1, a_s, a_d)


def _tc_mid(outp, denp, s1, d1, h1, b1r, W2, as2, ad2):
    def body(op_ref, dp_ref, s_ref, d_ref, h_ref, b_ref, w_ref, as_ref, ad_ref,
             h2_ref, s2_ref, d2_ref):
        sd = s_ref[...] + d_ref[...]
        exs = jnp.exp(jnp.maximum(sd, sd * 0.2))
        hprev = h_ref[...]
        outr = op_ref[0, :N, :] + op_ref[1, :N, :] + exs * hprev
        den = dp_ref[0, :N, :] + dp_ref[1, :N, :] + exs + 1e-16
        hmid = jnp.maximum(outr / den + b_ref[...], 0.0)
        h2 = jnp.dot(hmid, w_ref[...], preferred_element_type=jnp.float32)
        h2_ref[...] = h2
        s2_ref[...] = jnp.dot(h2, as_ref[...], preferred_element_type=jnp.float32)
        d2_ref[...] = jnp.dot(h2, ad_ref[...], preferred_element_type=jnp.float32)

    return pl.pallas_call(
        body,
        out_shape=(
            jax.ShapeDtypeStruct((N, H), jnp.float32),
            jax.ShapeDtypeStruct((N, 1), jnp.float32),
            jax.ShapeDtypeStruct((N, 1), jnp.float32),
        ),
        compiler_params=pltpu.CompilerParams(vmem_limit_bytes=100 * 1024 * 1024),
    )(outp, denp, s1, d1, h1, b1r, W2, as2, ad2)


def _tc_fin(outp, denp, s2, d2, h2, b2r, Wr, brr, Wc, bcr):
    def body(op_ref, dp_ref, s_ref, d_ref, h_ref, b_ref, wr_ref, br_ref,
             wc_ref, bc_ref, y_ref):
        sd = s_ref[...] + d_ref[...]
        exs = jnp.exp(jnp.maximum(sd, sd * 0.2))
        hprev = h_ref[...]
        outr = op_ref[0, :N, :] + op_ref[1, :N, :] + exs * hprev
        den = dp_ref[0, :N, :] + dp_ref[1, :N, :] + exs + 1e-16
        hmid = jnp.maximum(outr / den + b_ref[...], 0.0)
        t = jnp.dot(hmid, wr_ref[...], preferred_element_type=jnp.float32)
        t = t + br_ref[...]
        y = jnp.sum(t * wc_ref[...], axis=0, keepdims=True) + bc_ref[...]
        y_ref[...] = y

    return pl.pallas_call(
        body,
        out_shape=jax.ShapeDtypeStruct((1, C), jnp.float32),
        compiler_params=pltpu.CompilerParams(vmem_limit_bytes=100 * 1024 * 1024),
    )(outp, denp, s2, d2, h2, b2r, Wr, brr, Wc, bcr)


def kernel(x, edge_index, W1, a_src1, a_dst1, b1, W2, a_src2, a_dst2, b2,
           Wr, br, Wc, bc):
    src = edge_index[0]
    dst = edge_index[1]
    srcr = jnp.pad(src, (0, EP - E)).reshape(NW, NCH, K)
    dstr = jnp.pad(dst, (0, EP - E)).reshape(NW, NCH, K)

    h1, s1, d1 = _tc_pre(x, W1, a_src1.reshape(H, 1), a_dst1.reshape(H, 1))
    outp1, denp1 = _sc_gat_aggregate(h1, s1.reshape(N), d1.reshape(N), srcr, dstr)
    h2, s2, d2 = _tc_mid(outp1, denp1.reshape(NC, NP, 1), s1, d1, h1,
                         b1.reshape(1, H), W2,
                         a_src2.reshape(H, 1), a_dst2.reshape(H, 1))
    outp2, denp2 = _sc_gat_aggregate(h2, s2.reshape(N), d2.reshape(N), srcr, dstr)
    return _tc_fin(outp2, denp2.reshape(NC, NP, 1), s2, d2, h2,
                   b2.reshape(1, H), Wr, br.reshape(1, 1), Wc, bc.reshape(1, C))


# speedup vs baseline: 29.1710x; 1.0007x over previous
import functools

import jax
import jax.numpy as jnp
from jax import lax
from jax.experimental import pallas as pl
from jax.experimental.pallas import tpu as pltpu
from jax.experimental.pallas import tpu_sc as plsc

N = 10000
E = 320000
D = 128
H = 64
C = 2

NC = 2
NS = 16
NW = NC * NS
NP = 10240
EPT = 10240
EP = EPT * NW
K = 128
NCH = EPT // K
NSL = NP // NS


def _sc_gat_aggregate(h, asv, adv, srcr, dstr):
    mesh = plsc.VectorSubcoreMesh(core_axis_name="c", subcore_axis_name="s")

    @functools.partial(
        pl.kernel,
        out_type=(
            jax.ShapeDtypeStruct((NC, NP, H), jnp.float32),
            jax.ShapeDtypeStruct((NC, NP), jnp.float32),
        ),
        mesh=mesh,
        compiler_params=pltpu.CompilerParams(
            needs_layout_passes=False, use_tc_tiling_on_sc=False),
        scratch_types=[
            pltpu.VMEM((NCH, K), jnp.int32),
            pltpu.VMEM((NCH, K), jnp.int32),
            pltpu.VMEM((N,), jnp.float32),
            pltpu.VMEM((N,), jnp.float32),
            pltpu.VMEM((NCH, K), jnp.float32),
            pltpu.VMEM((2, K, H), jnp.float32),
            pltpu.VMEM((NSL,), jnp.float32),
            pltpu.VMEM_SHARED((NP, H), jnp.float32),
            pltpu.VMEM_SHARED((NP,), jnp.float32),
            pltpu.SemaphoreType.DMA,
            pltpu.SemaphoreType.DMA,
            pltpu.SemaphoreType.DMA,
        ],
    )
    def k(h_hbm, as_hbm, ad_hbm, src_hbm, dst_hbm,
          out_hbm, den_hbm,
          src_v, dst_v, as_v, ad_v, ex_v, rows_v, zden_v, acc_s, den_s,
          gsem, ssem, dsem):
        core = lax.axis_index("c")
        sid = lax.axis_index("s")
        wid = core * NS + sid

        pltpu.sync_copy(src_hbm.at[wid], src_v)
        pltpu.sync_copy(dst_hbm.at[wid], dst_v)
        pltpu.sync_copy(as_hbm, as_v)
        pltpu.sync_copy(ad_hbm, ad_v)

        z16 = jnp.zeros((16,), jnp.float32)

        def zrow(kk, carry):
            for j in range(H // 16):
                rows_v[0, kk, pl.ds(j * 16, 16)] = z16
            return carry
        lax.fori_loop(0, K, zrow, 0)

        def zden(i, carry):
            zden_v[pl.ds(i * 16, 16)] = z16
            return carry
        lax.fori_loop(0, NSL // 16, zden, 0)

        for q in range(NSL // K):
            pltpu.sync_copy(rows_v.at[0], acc_s.at[pl.ds(sid * NSL + q * K, K)])
        pltpu.sync_copy(zden_v, den_s.at[pl.ds(sid * NSL, NSL)])

        base = wid * EPT
        iota = lax.iota(jnp.int32, 16)

        def exbody(c, carry):
            for j in range(K // 16):
                s16 = src_v[c, pl.ds(j * 16, 16)]
                d16 = dst_v[c, pl.ds(j * 16, 16)]
                av = plsc.load_gather(as_v, [s16])
                bv = plsc.load_gather(ad_v, [d16])
                e = av + bv
                e = jnp.maximum(e, e * 0.2)
                ex = jnp.exp(e)
                gid = base + c * K + j * 16 + iota
                ex = jnp.where(gid < E, ex, 0.0)
                ex_v[c, pl.ds(j * 16, 16)] = ex
            return carry
        lax.fori_loop(0, NCH, exbody, 0)

        plsc.subcore_barrier()

        pltpu.async_copy(h_hbm.at[src_v.at[0]], rows_v.at[0], gsem)

        def chunk(c, carry):
            p = lax.rem(c, 2)
            pltpu.make_async_copy(h_hbm.at[src_v.at[c]], rows_v.at[p], gsem).wait()

            @pl.when(c >= 1)
            def _():
                pltpu.make_async_copy(
                    rows_v.at[1 - p], acc_s.at[dst_v.at[c - 1]], ssem).wait()

            @pl.when(c < NCH - 1)
            def _():
                pltpu.async_copy(h_hbm.at[src_v.at[c + 1]], rows_v.at[1 - p], gsem)

            def scale(kk, carry2):
                exb = plsc.load_gather(
                    ex_v, [jnp.full((16,), c, jnp.int32),
                           jnp.full((16,), kk, jnp.int32)])
                for j in range(H // 16):
                    rows_v[p, kk, pl.ds(j * 16, 16)] = (
                        rows_v[p, kk, pl.ds(j * 16, 16)] * exb)
                return carry2
            lax.fori_loop(0, K, scale, 0)

            pltpu.make_async_copy(
                rows_v.at[p], acc_s.at[dst_v.at[c]], ssem).start(add=True)
            pltpu.make_async_copy(
                ex_v.at[c], den_s.at[dst_v.at[c]], dsem).start(add=True)
            return carry
        lax.fori_loop(0, NCH, chunk, 0)

        pltpu.make_async_copy(
            rows_v.at[1], acc_s.at[dst_v.at[NCH - 1]], ssem).wait()

        def drain(c, carry):
            pltpu.make_async_copy(
                ex_v.at[0], den_s.at[dst_v.at[0]], dsem).wait()
            return carry
        lax.fori_loop(0, NCH, drain, 0)

        plsc.subcore_barrier()

        pltpu.sync_copy(acc_s.at[pl.ds(sid * NSL, NSL)],
                        out_hbm.at[core, pl.ds(sid * NSL, NSL)])
        pltpu.sync_copy(den_s.at[pl.ds(sid * NSL, NSL)],
                        den_hbm.at[core, pl.ds(sid * NSL, NSL)])

    return k(h, asv, adv, srcr, dstr)


def _tc_pre(x, W1, a_s, a_d):
    def body(x_ref, w_ref, as_ref, ad_ref, h_ref, s_ref, d_ref):
        h = jnp.dot(x_ref[...], w_ref[...], preferred_element_type=jnp.float32)
        h_ref[...] = h
        s_ref[...] = jnp.dot(h, as_ref[...], preferred_element_type=jnp.float32)
        d_ref[...] = jnp.dot(h, ad_ref[...], preferred_element_type=jnp.float32)

    return pl.pallas_call(
        body,
        out_shape=(
            jax.ShapeDtypeStruct((N, H), jnp.float32),
            jax.ShapeDtypeStruct((N, 1), jnp.float32),
            jax.ShapeDtypeStruct((N, 1), jnp.float32),
        ),
    )(x, W1, a_s, a_d)


def _tc_mid(outp, denp, s1, d1, h1, b1r, W2, as2, ad2):
    def body(op_ref, dp_ref, s_ref, d_ref, h_ref, b_ref, w_ref, as_ref, ad_ref,
             h2_ref, s2_ref, d2_ref):
        sd = s_ref[...] + d_ref[...]
        exs = jnp.exp(jnp.maximum(sd, sd * 0.2))
        hprev = h_ref[...]
        outr = op_ref[0, :N, :] + op_ref[1, :N, :] + exs * hprev
        den = dp_ref[0, :N, :] + dp_ref[1, :N, :] + exs + 1e-16
        hmid = jnp.maximum(outr / den + b_ref[...], 0.0)
        h2 = jnp.dot(hmid, w_ref[...], preferred_element_type=jnp.float32)
        h2_ref[...] = h2
        s2_ref[...] = jnp.dot(h2, as_ref[...], preferred_element_type=jnp.float32)
        d2_ref[...] = jnp.dot(h2, ad_ref[...], preferred_element_type=jnp.float32)

    return pl.pallas_call(
        body,
        out_shape=(
            jax.ShapeDtypeStruct((N, H), jnp.float32),
            jax.ShapeDtypeStruct((N, 1), jnp.float32),
            jax.ShapeDtypeStruct((N, 1), jnp.float32),
        ),
        compiler_params=pltpu.CompilerParams(vmem_limit_bytes=100 * 1024 * 1024),
    )(outp, denp, s1, d1, h1, b1r, W2, as2, ad2)


def _tc_fin(outp, denp, s2, d2, h2, b2r, Wr, brr, Wc, bcr):
    def body(op_ref, dp_ref, s_ref, d_ref, h_ref, b_ref, wr_ref, br_ref,
             wc_ref, bc_ref, y_ref):
        sd = s_ref[...] + d_ref[...]
        exs = jnp.exp(jnp.maximum(sd, sd * 0.2))
        hprev = h_ref[...]
        outr = op_ref[0, :N, :] + op_ref[1, :N, :] + exs * hprev
        den = dp_ref[0, :N, :] + dp_ref[1, :N, :] + exs + 1e-16
        hmid = jnp.maximum(outr / den + b_ref[...], 0.0)
        t = jnp.dot(hmid, wr_ref[...], preferred_element_type=jnp.float32)
        t = t + br_ref[...]
        y = jnp.sum(t * wc_ref[...], axis=0, keepdims=True) + bc_ref[...]
        y_ref[...] = y

    return pl.pallas_call(
        body,
        out_shape=jax.ShapeDtypeStruct((1, C), jnp.float32),
        compiler_params=pltpu.CompilerParams(vmem_limit_bytes=100 * 1024 * 1024),
    )(outp, denp, s2, d2, h2, b2r, Wr, brr, Wc, bcr)


def kernel(x, edge_index, W1, a_src1, a_dst1, b1, W2, a_src2, a_dst2, b2,
           Wr, br, Wc, bc):
    src = edge_index[0]
    dst = edge_index[1]
    srcr = jnp.pad(src, (0, EP - E)).reshape(NW, NCH, K)
    dstr = jnp.pad(dst, (0, EP - E)).reshape(NW, NCH, K)

    h1, s1, d1 = _tc_pre(x, W1, a_src1.reshape(H, 1), a_dst1.reshape(H, 1))
    outp1, denp1 = _sc_gat_aggregate(h1, s1.reshape(N), d1.reshape(N), srcr, dstr)
    h2, s2, d2 = _tc_mid(outp1, denp1.reshape(NC, NP, 1), s1, d1, h1,
                         b1.reshape(1, H), W2,
                         a_src2.reshape(H, 1), a_dst2.reshape(H, 1))
    outp2, denp2 = _sc_gat_aggregate(h2, s2.reshape(N), d2.reshape(N), srcr, dstr)
    return _tc_fin(outp2, denp2.reshape(NC, NP, 1), s2, d2, h2,
                   b2.reshape(1, H), Wr, br.reshape(1, 1), Wc, bc.reshape(1, C))


# speedup vs baseline: 40.4092x; 1.3853x over previous
import functools

import jax
import jax.numpy as jnp
from jax import lax
from jax.experimental import pallas as pl
from jax.experimental.pallas import tpu as pltpu
from jax.experimental.pallas import tpu_sc as plsc

N = 10000
E = 320000
D = 128
H = 64
HH = H // 2
C = 2

NC = 2
NS = 16
NP = 10240
EPT = 20480
EP = EPT * NS
K = 128
NCH = EPT // K
NSL = NP // NS
HSL = N // NS


def _sc_gat_aggregate(hsp, asv, adv, srcr, dstr):
    mesh = plsc.VectorSubcoreMesh(core_axis_name="c", subcore_axis_name="s")

    @functools.partial(
        pl.kernel,
        out_type=(
            jax.ShapeDtypeStruct((NC, N, HH), jnp.float32),
            jax.ShapeDtypeStruct((NP,), jnp.float32),
        ),
        mesh=mesh,
        compiler_params=pltpu.CompilerParams(
            needs_layout_passes=False, use_tc_tiling_on_sc=False),
        scratch_types=[
            pltpu.VMEM((NCH, K), jnp.int32),
            pltpu.VMEM((NCH, K), jnp.int32),
            pltpu.VMEM((N,), jnp.float32),
            pltpu.VMEM((N,), jnp.float32),
            pltpu.VMEM((NCH, K), jnp.float32),
            pltpu.VMEM((2, K, HH), jnp.float32),
            pltpu.VMEM((NSL,), jnp.float32),
            pltpu.VMEM_SHARED((N, HH), jnp.float32),
            pltpu.VMEM_SHARED((N, HH), jnp.float32),
            pltpu.VMEM_SHARED((NP,), jnp.float32),
            pltpu.SemaphoreType.DMA,
            pltpu.SemaphoreType.DMA,
            pltpu.SemaphoreType.DMA,
        ],
    )
    def k(h_hbm, as_hbm, ad_hbm, src_hbm, dst_hbm,
          out_hbm, den_hbm,
          src_v, dst_v, as_v, ad_v, ex_v, rows_v, zden_v, h_s, acc_s, den_s,
          gsem, ssem, dsem):
        core = lax.axis_index("c")
        sid = lax.axis_index("s")

        pltpu.sync_copy(src_hbm.at[sid], src_v)
        pltpu.sync_copy(dst_hbm.at[sid], dst_v)
        pltpu.sync_copy(as_hbm, as_v)
        pltpu.sync_copy(ad_hbm, ad_v)
        pltpu.sync_copy(h_hbm.at[core, pl.ds(sid * HSL, HSL)],
                        h_s.at[pl.ds(sid * HSL, HSL)])

        z16 = jnp.zeros((16,), jnp.float32)

        def zrow(kk, carry):
            for j in range(HH // 16):
                rows_v[0, kk, pl.ds(j * 16, 16)] = z16
            return carry
        lax.fori_loop(0, K, zrow, 0)

        for q in range(5):
            pltpu.sync_copy(rows_v.at[0, pl.ds(0, 125)],
                            acc_s.at[pl.ds(sid * HSL + q * 125, 125)])

        @pl.when(core == 0)
        def _():
            def zden(i, carry):
                zden_v[pl.ds(i * 16, 16)] = z16
                return carry
            lax.fori_loop(0, NSL // 16, zden, 0)
            pltpu.sync_copy(zden_v, den_s.at[pl.ds(sid * NSL, NSL)])

        base = sid * EPT
        iota = lax.iota(jnp.int32, 16)

        def exbody(c, carry):
            for j in range(K // 16):
                s16 = src_v[c, pl.ds(j * 16, 16)]
                d16 = dst_v[c, pl.ds(j * 16, 16)]
                av = plsc.load_gather(as_v, [s16])
                bv = plsc.load_gather(ad_v, [d16])
                e = av + bv
                e = jnp.maximum(e, e * 0.2)
                ex = jnp.exp(e)
                gid = base + c * K + j * 16 + iota
                ex = jnp.where(gid < E, ex, 0.0)
                ex_v[c, pl.ds(j * 16, 16)] = ex
            return carry
        lax.fori_loop(0, NCH, exbody, 0)

        plsc.subcore_barrier()

        pltpu.async_copy(h_s.at[src_v.at[0]], rows_v.at[0], gsem)

        def chunk(c, carry):
            p = lax.rem(c, 2)
            pltpu.make_async_copy(h_s.at[src_v.at[c]], rows_v.at[p], gsem).wait()

            @pl.when(c >= 1)
            def _():
                pltpu.make_async_copy(
                    rows_v.at[1 - p], acc_s.at[dst_v.at[c - 1]], ssem).wait()

            @pl.when(c < NCH - 1)
            def _():
                pltpu.async_copy(h_s.at[src_v.at[c + 1]], rows_v.at[1 - p], gsem)

            def scale(kk, carry2):
                exb = plsc.load_gather(
                    ex_v, [jnp.full((16,), c, jnp.int32),
                           jnp.full((16,), kk, jnp.int32)])
                for j in range(HH // 16):
                    rows_v[p, kk, pl.ds(j * 16, 16)] = (
                        rows_v[p, kk, pl.ds(j * 16, 16)] * exb)
                return carry2
            lax.fori_loop(0, K, scale, 0)

            pltpu.make_async_copy(
                rows_v.at[p], acc_s.at[dst_v.at[c]], ssem).start(add=True)

            @pl.when(core == 0)
            def _():
                pltpu.make_async_copy(
                    ex_v.at[c], den_s.at[dst_v.at[c]], dsem).start(add=True)
            return carry
        lax.fori_loop(0, NCH, chunk, 0)

        pltpu.make_async_copy(
            rows_v.at[1], acc_s.at[dst_v.at[NCH - 1]], ssem).wait()

        @pl.when(core == 0)
        def _():
            def drain(c, carry):
                pltpu.make_async_copy(
                    ex_v.at[0], den_s.at[dst_v.at[0]], dsem).wait()
                return carry
            lax.fori_loop(0, NCH, drain, 0)

        plsc.subcore_barrier()

        pltpu.sync_copy(acc_s.at[pl.ds(sid * HSL, HSL)],
                        out_hbm.at[core, pl.ds(sid * HSL, HSL)])

        @pl.when(core == 0)
        def _():
            pltpu.sync_copy(den_s.at[pl.ds(sid * NSL, NSL)],
                            den_hbm.at[pl.ds(sid * NSL, NSL)])

    return k(hsp, asv, adv, srcr, dstr)


def _tc_pre(x, W1, a_s, a_d):
    def body(x_ref, w_ref, as_ref, ad_ref, h_ref, s_ref, d_ref):
        h = jnp.dot(x_ref[...], w_ref[...], preferred_element_type=jnp.float32)
        h_ref[0, :, :] = h[:, :HH]
        h_ref[1, :, :] = h[:, HH:]
        s_ref[...] = jnp.dot(h, as_ref[...], preferred_element_type=jnp.float32)
        d_ref[...] = jnp.dot(h, ad_ref[...], preferred_element_type=jnp.float32)

    return pl.pallas_call(
        body,
        out_shape=(
            jax.ShapeDtypeStruct((2, N, HH), jnp.float32),
            jax.ShapeDtypeStruct((N, 1), jnp.float32),
            jax.ShapeDtypeStruct((N, 1), jnp.float32),
        ),
        compiler_params=pltpu.CompilerParams(vmem_limit_bytes=100 * 1024 * 1024),
    )(x, W1, a_s, a_d)


def _tc_mid(outp, denp, s1, d1, hsp, b1r, W2, as2, ad2):
    def body(op_ref, dp_ref, s_ref, d_ref, h_ref, b_ref, w_ref, as_ref, ad_ref,
             h2_ref, s2_ref, d2_ref):
        sd = s_ref[...] + d_ref[...]
        exs = jnp.exp(jnp.maximum(sd, sd * 0.2))
        den = dp_ref[...] + exs + 1e-16
        hmid0 = jnp.maximum(
            (op_ref[0, :, :] + exs * h_ref[0, :, :]) / den + b_ref[:, :HH], 0.0)
        hmid1 = jnp.maximum(
            (op_ref[1, :, :] + exs * h_ref[1, :, :]) / den + b_ref[:, HH:], 0.0)
        h2 = (jnp.dot(hmid0, w_ref[:HH, :], preferred_element_type=jnp.float32)
              + jnp.dot(hmid1, w_ref[HH:, :], preferred_element_type=jnp.float32))
        h2_ref[0, :, :] = h2[:, :HH]
        h2_ref[1, :, :] = h2[:, HH:]
        s2_ref[...] = jnp.dot(h2, as_ref[...], preferred_element_type=jnp.float32)
        d2_ref[...] = jnp.dot(h2, ad_ref[...], preferred_element_type=jnp.float32)

    NB = 2000
    row3 = pl.BlockSpec((2, NB, HH), lambda i: (0, i, 0))
    col1 = pl.BlockSpec((NB, 1), lambda i: (i, 0))
    full = lambda a: pl.BlockSpec(a.shape, lambda i: tuple(0 for _ in a.shape))
    return pl.pallas_call(
        body,
        grid=(N // NB,),
        in_specs=[row3, col1, col1, col1, row3,
                  full(b1r), full(W2), full(as2), full(ad2)],
        out_specs=(row3, col1, col1),
        out_shape=(
            jax.ShapeDtypeStruct((2, N, HH), jnp.float32),
            jax.ShapeDtypeStruct((N, 1), jnp.float32),
            jax.ShapeDtypeStruct((N, 1), jnp.float32),
        ),
        compiler_params=pltpu.CompilerParams(vmem_limit_bytes=100 * 1024 * 1024),
    )(outp, denp, s1, d1, hsp, b1r, W2, as2, ad2)


def _tc_fin(outp, denp, s2, d2, hsp, b2r, Wr, brr, Wc, bcr):
    def body(op_ref, dp_ref, s_ref, d_ref, h_ref, b_ref, wr_ref, br_ref,
             wc_ref, bc_ref, y_ref):
        sd = s_ref[...] + d_ref[...]
        exs = jnp.exp(jnp.maximum(sd, sd * 0.2))
        den = dp_ref[...] + exs + 1e-16
        hmid0 = jnp.maximum(
            (op_ref[0, :, :] + exs * h_ref[0, :, :]) / den + b_ref[:, :HH], 0.0)
        hmid1 = jnp.maximum(
            (op_ref[1, :, :] + exs * h_ref[1, :, :]) / den + b_ref[:, HH:], 0.0)
        t = (jnp.dot(hmid0, wr_ref[:HH, :], preferred_element_type=jnp.float32)
             + jnp.dot(hmid1, wr_ref[HH:, :], preferred_element_type=jnp.float32))
        t = t + br_ref[...]
        part = jnp.sum(t * wc_ref[...], axis=0, keepdims=True)
        i = pl.program_id(0)

        @pl.when(i == 0)
        def _():
            y_ref[...] = part + bc_ref[...]

        @pl.when(i > 0)
        def _():
            y_ref[...] = y_ref[...] + part

    NB = 2000
    row3 = pl.BlockSpec((2, NB, HH), lambda i: (0, i, 0))
    col1 = pl.BlockSpec((NB, 1), lambda i: (i, 0))
    colC = pl.BlockSpec((NB, C), lambda i: (i, 0))
    full = lambda a: pl.BlockSpec(a.shape, lambda i: tuple(0 for _ in a.shape))
    return pl.pallas_call(
        body,
        grid=(N // NB,),
        in_specs=[row3, col1, col1, col1, row3,
                  full(b2r), full(Wr), full(brr), colC, full(bcr)],
        out_specs=pl.BlockSpec((1, C), lambda i: (0, 0)),
        out_shape=jax.ShapeDtypeStruct((1, C), jnp.float32),
        compiler_params=pltpu.CompilerParams(vmem_limit_bytes=100 * 1024 * 1024),
    )(outp, denp, s2, d2, hsp, b2r, Wr, brr, Wc, bcr)


def kernel(x, edge_index, W1, a_src1, a_dst1, b1, W2, a_src2, a_dst2, b2,
           Wr, br, Wc, bc):
    src = edge_index[0]
    dst = edge_index[1]
    srcr = jnp.pad(src, (0, EP - E)).reshape(NS, NCH, K)
    dstr = jnp.pad(dst, (0, EP - E)).reshape(NS, NCH, K)

    hs1, s1, d1 = _tc_pre(x, W1, a_src1.reshape(H, 1), a_dst1.reshape(H, 1))
    outp1, den1 = _sc_gat_aggregate(hs1, s1.reshape(N), d1.reshape(N),
                                    srcr, dstr)
    hs2, s2, d2 = _tc_mid(outp1, den1[:N].reshape(N, 1), s1, d1, hs1,
                          b1.reshape(1, H), W2,
                          a_src2.reshape(H, 1), a_dst2.reshape(H, 1))
    outp2, den2 = _sc_gat_aggregate(hs2, s2.reshape(N), d2.reshape(N),
                                    srcr, dstr)
    return _tc_fin(outp2, den2[:N].reshape(N, 1), s2, d2, hs2,
                   b2.reshape(1, H), Wr, br.reshape(1, 1), Wc, bc.reshape(1, C))


# speedup vs baseline: 40.9029x; 1.0122x over previous
import functools

import jax
import jax.numpy as jnp
from jax import lax
from jax.experimental import pallas as pl
from jax.experimental.pallas import tpu as pltpu
from jax.experimental.pallas import tpu_sc as plsc

N = 10000
E = 320000
D = 128
H = 64
HH = H // 2
C = 2

NC = 2
NS = 16
NP = 10240
EPT = 20480
EP = EPT * NS
K = 128
NCH = EPT // K
NSL = NP // NS
HSL = N // NS


def _sc_gat_aggregate(hsp, asv, adv, srcr, dstr):
    mesh = plsc.VectorSubcoreMesh(core_axis_name="c", subcore_axis_name="s")

    @functools.partial(
        pl.kernel,
        out_type=(
            jax.ShapeDtypeStruct((NC, N, HH), jnp.float32),
            jax.ShapeDtypeStruct((NC, NP), jnp.float32),
        ),
        mesh=mesh,
        compiler_params=pltpu.CompilerParams(
            needs_layout_passes=False, use_tc_tiling_on_sc=False),
        scratch_types=[
            pltpu.VMEM((NCH, K), jnp.int32),
            pltpu.VMEM((NCH, K), jnp.int32),
            pltpu.VMEM((N,), jnp.float32),
            pltpu.VMEM((N,), jnp.float32),
            pltpu.VMEM((NCH, K), jnp.float32),
            pltpu.VMEM((2, K, HH), jnp.float32),
            pltpu.VMEM((NSL,), jnp.float32),
            pltpu.VMEM_SHARED((N, HH), jnp.float32),
            pltpu.VMEM_SHARED((N, HH), jnp.float32),
            pltpu.VMEM_SHARED((NP,), jnp.float32),
            pltpu.SemaphoreType.DMA,
            pltpu.SemaphoreType.DMA,
            pltpu.SemaphoreType.DMA,
        ],
    )
    def k(h_hbm, as_hbm, ad_hbm, src_hbm, dst_hbm,
          out_hbm, den_hbm,
          src_v, dst_v, as_v, ad_v, ex_v, rows_v, zden_v, h_s, acc_s, den_s,
          gsem, ssem, dsem):
        core = lax.axis_index("c")
        sid = lax.axis_index("s")

        pltpu.sync_copy(src_hbm.at[sid], src_v)
        pltpu.sync_copy(dst_hbm.at[sid], dst_v)
        pltpu.sync_copy(as_hbm, as_v)
        pltpu.sync_copy(ad_hbm, ad_v)
        pltpu.sync_copy(h_hbm.at[core, pl.ds(sid * HSL, HSL)],
                        h_s.at[pl.ds(sid * HSL, HSL)])

        z16 = jnp.zeros((16,), jnp.float32)

        def zrow(kk, carry):
            for j in range(HH // 16):
                rows_v[0, kk, pl.ds(j * 16, 16)] = z16
            return carry
        lax.fori_loop(0, K, zrow, 0)

        for q in range(5):
            pltpu.sync_copy(rows_v.at[0, pl.ds(0, 125)],
                            acc_s.at[pl.ds(sid * HSL + q * 125, 125)])

        def zden(i, carry):
            zden_v[pl.ds(i * 16, 16)] = z16
            return carry
        lax.fori_loop(0, NSL // 16, zden, 0)
        pltpu.sync_copy(zden_v, den_s.at[pl.ds(sid * NSL, NSL)])

        base = sid * EPT
        iota = lax.iota(jnp.int32, 16)

        def exbody(c, carry):
            for j in range(K // 16):
                s16 = src_v[c, pl.ds(j * 16, 16)]
                d16 = dst_v[c, pl.ds(j * 16, 16)]
                av = plsc.load_gather(as_v, [s16])
                bv = plsc.load_gather(ad_v, [d16])
                e = av + bv
                e = jnp.maximum(e, e * 0.2)
                ex = jnp.exp(e)
                gid = base + c * K + j * 16 + iota
                ex = jnp.where(gid < E, ex, 0.0)
                ex_v[c, pl.ds(j * 16, 16)] = ex
            return carry
        lax.fori_loop(0, NCH, exbody, 0)

        plsc.subcore_barrier()

        pltpu.async_copy(h_s.at[src_v.at[0]], rows_v.at[0], gsem)

        def chunk(c, carry):
            p = lax.rem(c, 2)
            pltpu.make_async_copy(h_s.at[src_v.at[c]], rows_v.at[p], gsem).wait()

            @pl.when(c >= 1)
            def _():
                pltpu.make_async_copy(
                    rows_v.at[0], acc_s.at[dst_v.at[0]], ssem).wait()

            @pl.when(c < NCH - 1)
            def _():
                pltpu.async_copy(
                    h_s.at[src_v.at[c + 1]], rows_v.at[lax.rem(c + 1, 2)], gsem)

            def scale(kk, carry2):
                exb = plsc.load_gather(
                    ex_v, [jnp.full((16,), c, jnp.int32),
                           jnp.full((16,), kk, jnp.int32)])
                for j in range(HH // 16):
                    rows_v[p, kk, pl.ds(j * 16, 16)] = (
                        rows_v[p, kk, pl.ds(j * 16, 16)] * exb)
                return carry2
            lax.fori_loop(0, K, scale, 0)

            pltpu.make_async_copy(
                rows_v.at[p], acc_s.at[dst_v.at[c]], ssem).start(add=True)

            do_den = jnp.where(core == 0, c < NCH // 2, c >= NCH // 2)

            @pl.when(do_den)
            def _():
                pltpu.make_async_copy(
                    ex_v.at[c], den_s.at[dst_v.at[c]], dsem).start(add=True)
            return carry
        lax.fori_loop(0, NCH, chunk, 0)

        pltpu.make_async_copy(
            rows_v.at[0], acc_s.at[dst_v.at[0]], ssem).wait()

        def drain(c, carry):
            pltpu.make_async_copy(
                ex_v.at[0], den_s.at[dst_v.at[0]], dsem).wait()
            return carry
        lax.fori_loop(0, NCH // 2, drain, 0)

        plsc.subcore_barrier()

        pltpu.sync_copy(acc_s.at[pl.ds(sid * HSL, HSL)],
                        out_hbm.at[core, pl.ds(sid * HSL, HSL)])

        pltpu.sync_copy(den_s.at[pl.ds(sid * NSL, NSL)],
                        den_hbm.at[core, pl.ds(sid * NSL, NSL)])

    return k(hsp, asv, adv, srcr, dstr)


def _tc_pre(x, W1, a_s, a_d):
    def body(x_ref, w_ref, as_ref, ad_ref, h_ref, s_ref, d_ref):
        h = jnp.dot(x_ref[...], w_ref[...], preferred_element_type=jnp.float32)
        h_ref[0, :, :] = h[:, :HH]
        h_ref[1, :, :] = h[:, HH:]
        s_ref[...] = jnp.dot(h, as_ref[...], preferred_element_type=jnp.float32)
        d_ref[...] = jnp.dot(h, ad_ref[...], preferred_element_type=jnp.float32)

    return pl.pallas_call(
        body,
        out_shape=(
            jax.ShapeDtypeStruct((2, N, HH), jnp.float32),
            jax.ShapeDtypeStruct((N, 1), jnp.float32),
            jax.ShapeDtypeStruct((N, 1), jnp.float32),
        ),
        compiler_params=pltpu.CompilerParams(vmem_limit_bytes=100 * 1024 * 1024),
    )(x, W1, a_s, a_d)


def _tc_mid(outp, denp, s1, d1, hsp, b1r, W2, as2, ad2):
    def body(op_ref, dp_ref, s_ref, d_ref, h_ref, b_ref, w_ref, as_ref, ad_ref,
             h2_ref, s2_ref, d2_ref):
        sd = s_ref[...] + d_ref[...]
        exs = jnp.exp(jnp.maximum(sd, sd * 0.2))
        den = dp_ref[0, :, :] + dp_ref[1, :, :] + exs + 1e-16
        hmid0 = jnp.maximum(
            (op_ref[0, :, :] + exs * h_ref[0, :, :]) / den + b_ref[:, :HH], 0.0)
        hmid1 = jnp.maximum(
            (op_ref[1, :, :] + exs * h_ref[1, :, :]) / den + b_ref[:, HH:], 0.0)
        h2 = (jnp.dot(hmid0, w_ref[:HH, :], preferred_element_type=jnp.float32)
              + jnp.dot(hmid1, w_ref[HH:, :], preferred_element_type=jnp.float32))
        h2_ref[0, :, :] = h2[:, :HH]
        h2_ref[1, :, :] = h2[:, HH:]
        s2_ref[...] = jnp.dot(h2, as_ref[...], preferred_element_type=jnp.float32)
        d2_ref[...] = jnp.dot(h2, ad_ref[...], preferred_element_type=jnp.float32)

    NB = 2000
    row3 = pl.BlockSpec((2, NB, HH), lambda i: (0, i, 0))
    col1 = pl.BlockSpec((NB, 1), lambda i: (i, 0))
    den3 = pl.BlockSpec((2, NB, 1), lambda i: (0, i, 0))
    full = lambda a: pl.BlockSpec(a.shape, lambda i: tuple(0 for _ in a.shape))
    return pl.pallas_call(
        body,
        grid=(N // NB,),
        in_specs=[row3, den3, col1, col1, row3,
                  full(b1r), full(W2), full(as2), full(ad2)],
        out_specs=(row3, col1, col1),
        out_shape=(
            jax.ShapeDtypeStruct((2, N, HH), jnp.float32),
            jax.ShapeDtypeStruct((N, 1), jnp.float32),
            jax.ShapeDtypeStruct((N, 1), jnp.float32),
        ),
        compiler_params=pltpu.CompilerParams(vmem_limit_bytes=100 * 1024 * 1024),
    )(outp, denp, s1, d1, hsp, b1r, W2, as2, ad2)


def _tc_fin(outp, denp, s2, d2, hsp, b2r, Wr, brr, Wc, bcr):
    def body(op_ref, dp_ref, s_ref, d_ref, h_ref, b_ref, wr_ref, br_ref,
             wc_ref, bc_ref, y_ref):
        sd = s_ref[...] + d_ref[...]
        exs = jnp.exp(jnp.maximum(sd, sd * 0.2))
        den = dp_ref[0, :, :] + dp_ref[1, :, :] + exs + 1e-16
        hmid0 = jnp.maximum(
            (op_ref[0, :, :] + exs * h_ref[0, :, :]) / den + b_ref[:, :HH], 0.0)
        hmid1 = jnp.maximum(
            (op_ref[1, :, :] + exs * h_ref[1, :, :]) / den + b_ref[:, HH:], 0.0)
        t = (jnp.dot(hmid0, wr_ref[:HH, :], preferred_element_type=jnp.float32)
             + jnp.dot(hmid1, wr_ref[HH:, :], preferred_element_type=jnp.float32))
        t = t + br_ref[...]
        part = jnp.sum(t * wc_ref[...], axis=0, keepdims=True)
        i = pl.program_id(0)

        @pl.when(i == 0)
        def _():
            y_ref[...] = part + bc_ref[...]

        @pl.when(i > 0)
        def _():
            y_ref[...] = y_ref[...] + part

    NB = 2000
    row3 = pl.BlockSpec((2, NB, HH), lambda i: (0, i, 0))
    col1 = pl.BlockSpec((NB, 1), lambda i: (i, 0))
    colC = pl.BlockSpec((NB, C), lambda i: (i, 0))
    den3 = pl.BlockSpec((2, NB, 1), lambda i: (0, i, 0))
    full = lambda a: pl.BlockSpec(a.shape, lambda i: tuple(0 for _ in a.shape))
    return pl.pallas_call(
        body,
        grid=(N // NB,),
        in_specs=[row3, den3, col1, col1, row3,
                  full(b2r), full(Wr), full(brr), colC, full(bcr)],
        out_specs=pl.BlockSpec((1, C), lambda i: (0, 0)),
        out_shape=jax.ShapeDtypeStruct((1, C), jnp.float32),
        compiler_params=pltpu.CompilerParams(vmem_limit_bytes=100 * 1024 * 1024),
    )(outp, denp, s2, d2, hsp, b2r, Wr, brr, Wc, bcr)


def kernel(x, edge_index, W1, a_src1, a_dst1, b1, W2, a_src2, a_dst2, b2,
           Wr, br, Wc, bc):
    src = edge_index[0]
    dst = edge_index[1]
    srcr = jnp.pad(src, (0, EP - E)).reshape(NS, NCH, K)
    dstr = jnp.pad(dst, (0, EP - E)).reshape(NS, NCH, K)

    hs1, s1, d1 = _tc_pre(x, W1, a_src1.reshape(H, 1), a_dst1.reshape(H, 1))
    outp1, den1 = _sc_gat_aggregate(hs1, s1.reshape(N), d1.reshape(N),
                                    srcr, dstr)
    hs2, s2, d2 = _tc_mid(outp1, den1[:, :N].reshape(NC, N, 1), s1, d1, hs1,
                          b1.reshape(1, H), W2,
                          a_src2.reshape(H, 1), a_dst2.reshape(H, 1))
    outp2, den2 = _sc_gat_aggregate(hs2, s2.reshape(N), d2.reshape(N),
                                    srcr, dstr)
    return _tc_fin(outp2, den2[:, :N].reshape(NC, N, 1), s2, d2, hs2,
                   b2.reshape(1, H), Wr, br.reshape(1, 1), Wc, bc.reshape(1, C))


# speedup vs baseline: 44.7783x; 1.0947x over previous
import functools

import jax
import jax.numpy as jnp
from jax import lax
from jax.experimental import pallas as pl
from jax.experimental.pallas import tpu as pltpu
from jax.experimental.pallas import tpu_sc as plsc

N = 10000
E = 320000
D = 128
H = 64
HH = H // 2
C = 2

NC = 2
NS = 16
NP = 10240
EPT = 20480
EP = EPT * NS
K = 128
NCH = EPT // K
NSL = NP // NS
HSL = N // NS


def _sc_gat_aggregate(hsp, asv, adv, srcr, dstr):
    mesh = plsc.VectorSubcoreMesh(core_axis_name="c", subcore_axis_name="s")

    @functools.partial(
        pl.kernel,
        out_type=(
            jax.ShapeDtypeStruct((NC, N, HH), jnp.float32),
            jax.ShapeDtypeStruct((NC, NP), jnp.float32),
        ),
        mesh=mesh,
        compiler_params=pltpu.CompilerParams(
            needs_layout_passes=False, use_tc_tiling_on_sc=False),
        scratch_types=[
            pltpu.VMEM((NCH, K), jnp.int32),
            pltpu.VMEM((NCH, K), jnp.int32),
            pltpu.VMEM((N,), jnp.float32),
            pltpu.VMEM((N,), jnp.float32),
            pltpu.VMEM((NCH, K), jnp.float32),
            pltpu.VMEM((2, K, HH), jnp.float32),
            pltpu.VMEM((NSL,), jnp.float32),
            pltpu.VMEM_SHARED((N, HH), jnp.float32),
            pltpu.VMEM_SHARED((N, HH), jnp.float32),
            pltpu.VMEM_SHARED((NP,), jnp.float32),
            pltpu.SemaphoreType.DMA,
            pltpu.SemaphoreType.DMA,
            pltpu.SemaphoreType.DMA,
        ],
    )
    def k(h_hbm, as_hbm, ad_hbm, src_hbm, dst_hbm,
          out_hbm, den_hbm,
          src_v, dst_v, as_v, ad_v, ex_v, rows_v, zden_v, h_s, acc_s, den_s,
          gsem, ssem, dsem):
        core = lax.axis_index("c")
        sid = lax.axis_index("s")

        pltpu.sync_copy(src_hbm.at[sid], src_v)
        pltpu.sync_copy(dst_hbm.at[sid], dst_v)
        pltpu.sync_copy(as_hbm, as_v)
        pltpu.sync_copy(ad_hbm, ad_v)
        pltpu.sync_copy(h_hbm.at[core, pl.ds(sid * HSL, HSL)],
                        h_s.at[pl.ds(sid * HSL, HSL)])

        z16 = jnp.zeros((16,), jnp.float32)

        def zrow(kk, carry):
            for j in range(HH // 16):
                rows_v[0, kk, pl.ds(j * 16, 16)] = z16
            return carry
        lax.fori_loop(0, K, zrow, 0)

        for q in range(5):
            pltpu.sync_copy(rows_v.at[0, pl.ds(0, 125)],
                            acc_s.at[pl.ds(sid * HSL + q * 125, 125)])

        def zden(i, carry):
            zden_v[pl.ds(i * 16, 16)] = z16
            return carry
        lax.fori_loop(0, NSL // 16, zden, 0)
        pltpu.sync_copy(zden_v, den_s.at[pl.ds(sid * NSL, NSL)])

        base = sid * EPT
        iota = lax.iota(jnp.int32, 16)

        def ex_chunk(c):
            for j in range(K // 16):
                s16 = src_v[c, pl.ds(j * 16, 16)]
                d16 = dst_v[c, pl.ds(j * 16, 16)]
                av = plsc.load_gather(as_v, [s16])
                bv = plsc.load_gather(ad_v, [d16])
                e = av + bv
                e = jnp.maximum(e, e * 0.2)
                ex = jnp.exp(e)
                gid = base + c * K + j * 16 + iota
                ex = jnp.where(gid < E, ex, 0.0)
                ex_v[c, pl.ds(j * 16, 16)] = ex

        plsc.subcore_barrier()

        pltpu.async_copy(h_s.at[src_v.at[0]], rows_v.at[0], gsem)

        def chunk(c, carry):
            p = lax.rem(c, 2)
            ex_chunk(c)
            pltpu.make_async_copy(h_s.at[src_v.at[c]], rows_v.at[p], gsem).wait()

            @pl.when(c >= 1)
            def _():
                pltpu.make_async_copy(
                    rows_v.at[0], acc_s.at[dst_v.at[0]], ssem).wait()

            @pl.when(c < NCH - 1)
            def _():
                pltpu.async_copy(
                    h_s.at[src_v.at[c + 1]], rows_v.at[lax.rem(c + 1, 2)], gsem)

            def scale(kk, carry2):
                exb = plsc.load_gather(
                    ex_v, [jnp.full((16,), c, jnp.int32),
                           jnp.full((16,), kk, jnp.int32)])
                for j in range(HH // 16):
                    rows_v[p, kk, pl.ds(j * 16, 16)] = (
                        rows_v[p, kk, pl.ds(j * 16, 16)] * exb)
                return carry2
            lax.fori_loop(0, K, scale, 0)

            pltpu.make_async_copy(
                rows_v.at[p], acc_s.at[dst_v.at[c]], ssem).start(add=True)

            do_den = jnp.where(core == 0, c < NCH // 2, c >= NCH // 2)

            @pl.when(do_den)
            def _():
                pltpu.make_async_copy(
                    ex_v.at[c], den_s.at[dst_v.at[c]], dsem).start(add=True)
            return carry
        lax.fori_loop(0, NCH, chunk, 0)

        pltpu.make_async_copy(
            rows_v.at[0], acc_s.at[dst_v.at[0]], ssem).wait()

        def drain(c, carry):
            pltpu.make_async_copy(
                ex_v.at[0], den_s.at[dst_v.at[0]], dsem).wait()
            return carry
        lax.fori_loop(0, NCH // 2, drain, 0)

        plsc.subcore_barrier()

        pltpu.sync_copy(acc_s.at[pl.ds(sid * HSL, HSL)],
                        out_hbm.at[core, pl.ds(sid * HSL, HSL)])

        pltpu.sync_copy(den_s.at[pl.ds(sid * NSL, NSL)],
                        den_hbm.at[core, pl.ds(sid * NSL, NSL)])

    return k(hsp, asv, adv, srcr, dstr)


def _tc_pre(x, W1, a_s, a_d):
    def body(x_ref, w_ref, as_ref, ad_ref, h_ref, s_ref, d_ref):
        h = jnp.dot(x_ref[...], w_ref[...], preferred_element_type=jnp.float32)
        h_ref[0, :, :] = h[:, :HH]
        h_ref[1, :, :] = h[:, HH:]
        s_ref[...] = jnp.dot(h, as_ref[...], preferred_element_type=jnp.float32)
        d_ref[...] = jnp.dot(h, ad_ref[...], preferred_element_type=jnp.float32)

    return pl.pallas_call(
        body,
        out_shape=(
            jax.ShapeDtypeStruct((2, N, HH), jnp.float32),
            jax.ShapeDtypeStruct((N, 1), jnp.float32),
            jax.ShapeDtypeStruct((N, 1), jnp.float32),
        ),
        compiler_params=pltpu.CompilerParams(vmem_limit_bytes=100 * 1024 * 1024),
    )(x, W1, a_s, a_d)


def _tc_mid(outp, denp, s1, d1, hsp, b1r, W2, as2, ad2):
    def body(op_ref, dp_ref, s_ref, d_ref, h_ref, b_ref, w_ref, as_ref, ad_ref,
             h2_ref, s2_ref, d2_ref):
        sd = s_ref[...] + d_ref[...]
        exs = jnp.exp(jnp.maximum(sd, sd * 0.2))
        den = dp_ref[0, :, :] + dp_ref[1, :, :] + exs + 1e-16
        hmid0 = jnp.maximum(
            (op_ref[0, :, :] + exs * h_ref[0, :, :]) / den + b_ref[:, :HH], 0.0)
        hmid1 = jnp.maximum(
            (op_ref[1, :, :] + exs * h_ref[1, :, :]) / den + b_ref[:, HH:], 0.0)
        h2 = (jnp.dot(hmid0, w_ref[:HH, :], preferred_element_type=jnp.float32)
              + jnp.dot(hmid1, w_ref[HH:, :], preferred_element_type=jnp.float32))
        h2_ref[0, :, :] = h2[:, :HH]
        h2_ref[1, :, :] = h2[:, HH:]
        s2_ref[...] = jnp.dot(h2, as_ref[...], preferred_element_type=jnp.float32)
        d2_ref[...] = jnp.dot(h2, ad_ref[...], preferred_element_type=jnp.float32)

    NB = 2000
    row3 = pl.BlockSpec((2, NB, HH), lambda i: (0, i, 0))
    col1 = pl.BlockSpec((NB, 1), lambda i: (i, 0))
    den3 = pl.BlockSpec((2, NB, 1), lambda i: (0, i, 0))
    full = lambda a: pl.BlockSpec(a.shape, lambda i: tuple(0 for _ in a.shape))
    return pl.pallas_call(
        body,
        grid=(N // NB,),
        in_specs=[row3, den3, col1, col1, row3,
                  full(b1r), full(W2), full(as2), full(ad2)],
        out_specs=(row3, col1, col1),
        out_shape=(
            jax.ShapeDtypeStruct((2, N, HH), jnp.float32),
            jax.ShapeDtypeStruct((N, 1), jnp.float32),
            jax.ShapeDtypeStruct((N, 1), jnp.float32),
        ),
        compiler_params=pltpu.CompilerParams(vmem_limit_bytes=100 * 1024 * 1024),
    )(outp, denp, s1, d1, hsp, b1r, W2, as2, ad2)


def _tc_fin(outp, denp, s2, d2, hsp, b2r, Wr, brr, Wc, bcr):
    def body(op_ref, dp_ref, s_ref, d_ref, h_ref, b_ref, wr_ref, br_ref,
             wc_ref, bc_ref, y_ref):
        sd = s_ref[...] + d_ref[...]
        exs = jnp.exp(jnp.maximum(sd, sd * 0.2))
        den = dp_ref[0, :, :] + dp_ref[1, :, :] + exs + 1e-16
        hmid0 = jnp.maximum(
            (op_ref[0, :, :] + exs * h_ref[0, :, :]) / den + b_ref[:, :HH], 0.0)
        hmid1 = jnp.maximum(
            (op_ref[1, :, :] + exs * h_ref[1, :, :]) / den + b_ref[:, HH:], 0.0)
        t = (jnp.dot(hmid0, wr_ref[:HH, :], preferred_element_type=jnp.float32)
             + jnp.dot(hmid1, wr_ref[HH:, :], preferred_element_type=jnp.float32))
        t = t + br_ref[...]
        part = jnp.sum(t * wc_ref[...], axis=0, keepdims=True)
        i = pl.program_id(0)

        @pl.when(i == 0)
        def _():
            y_ref[...] = part + bc_ref[...]

        @pl.when(i > 0)
        def _():
            y_ref[...] = y_ref[...] + part

    NB = 2000
    row3 = pl.BlockSpec((2, NB, HH), lambda i: (0, i, 0))
    col1 = pl.BlockSpec((NB, 1), lambda i: (i, 0))
    colC = pl.BlockSpec((NB, C), lambda i: (i, 0))
    den3 = pl.BlockSpec((2, NB, 1), lambda i: (0, i, 0))
    full = lambda a: pl.BlockSpec(a.shape, lambda i: tuple(0 for _ in a.shape))
    return pl.pallas_call(
        body,
        grid=(N // NB,),
        in_specs=[row3, den3, col1, col1, row3,
                  full(b2r), full(Wr), full(brr), colC, full(bcr)],
        out_specs=pl.BlockSpec((1, C), lambda i: (0, 0)),
        out_shape=jax.ShapeDtypeStruct((1, C), jnp.float32),
        compiler_params=pltpu.CompilerParams(vmem_limit_bytes=100 * 1024 * 1024),
    )(outp, denp, s2, d2, hsp, b2r, Wr, brr, Wc, bcr)


def kernel(x, edge_index, W1, a_src1, a_dst1, b1, W2, a_src2, a_dst2, b2,
           Wr, br, Wc, bc):
    src = edge_index[0]
    dst = edge_index[1]
    srcr = jnp.pad(src, (0, EP - E)).reshape(NS, NCH, K)
    dstr = jnp.pad(dst, (0, EP - E)).reshape(NS, NCH, K)

    hs1, s1, d1 = _tc_pre(x, W1, a_src1.reshape(H, 1), a_dst1.reshape(H, 1))
    outp1, den1 = _sc_gat_aggregate(hs1, s1.reshape(N), d1.reshape(N),
                                    srcr, dstr)
    hs2, s2, d2 = _tc_mid(outp1, den1[:, :N].reshape(NC, N, 1), s1, d1, hs1,
                          b1.reshape(1, H), W2,
                          a_src2.reshape(H, 1), a_dst2.reshape(H, 1))
    outp2, den2 = _sc_gat_aggregate(hs2, s2.reshape(N), d2.reshape(N),
                                    srcr, dstr)
    return _tc_fin(outp2, den2[:, :N].reshape(NC, N, 1), s2, d2, hs2,
                   b2.reshape(1, H), Wr, br.reshape(1, 1), Wc, bc.reshape(1, C))
